# Initial kernel scaffold; baseline (speedup 1.0000x reference)
import jax
import jax.numpy as jnp
from jax.experimental import pallas as pl


def kernel(inputs, embeddings_weight):
    raise NotImplementedError("write your pallas kernel here")


# speedup vs baseline: 1.3377x; 1.3377x over previous
import functools

import jax
import jax.numpy as jnp
from jax import lax
from jax.experimental import pallas as pl
from jax.experimental.pallas import tpu as pltpu


def _vq_body(x_ref, w_ref, q_ref, loss_ref, *, C):
    i = pl.program_id(0)
    xp = x_ref[0]
    xr = xp.T
    w = w_ref[...]

    x2 = jnp.sum(xr * xr, axis=1, keepdims=True)
    s = lax.dot_general(xr, w, (((1,), (1,)), ((), ())),
                        preferred_element_type=jnp.float32)
    w2 = jnp.sum(w * w, axis=1, keepdims=True)
    w2r = w2.T
    d2 = jnp.maximum(x2 - 2.0 * s + w2r, 0.0)
    dist = jnp.sqrt(d2)

    minv = jnp.min(dist, axis=1, keepdims=True)
    iota = lax.broadcasted_iota(jnp.int32, dist.shape, 1)
    idx = jnp.min(jnp.where(dist == minv, iota, C), axis=1,
                  keepdims=True)

    onehot = (iota == idx).astype(jnp.float32)
    q = lax.dot_general(onehot, w, (((1,), (0,)), ((), ())),
                        preferred_element_type=jnp.float32)

    qst = xr + (q - xr)
    q_ref[0] = qst.T

    diff = q - xr
    part = jnp.sum(diff * diff)

    @pl.when(i == 0)
    def _():
        loss_ref[0, 0] = 0.0

    loss_ref[0, 0] += part


def kernel(inputs, embeddings_weight):
    N, D, H, W = inputs.shape
    C = embeddings_weight.shape[0]
    P = H * W
    x3 = inputs.reshape(N, D, P)

    q3, loss_sum = pl.pallas_call(
        functools.partial(_vq_body, C=C),
        grid=(N,),
        in_specs=[
            pl.BlockSpec((1, D, P), lambda i: (i, 0, 0)),
            pl.BlockSpec((C, D), lambda i: (0, 0)),
        ],
        out_specs=[
            pl.BlockSpec((1, D, P), lambda i: (i, 0, 0)),
            pl.BlockSpec((1, 1), lambda i: (0, 0), memory_space=pltpu.SMEM),
        ],
        out_shape=[
            jax.ShapeDtypeStruct((N, D, P), jnp.float32),
            jax.ShapeDtypeStruct((1, 1), jnp.float32),
        ],
    )(x3, embeddings_weight)

    quantized_st = q3.reshape(N, D, H, W)
    c_loss = loss_sum[0, 0] * jnp.float32(1.25) / jnp.float32(N * D * H * W)
    return (c_loss, quantized_st)


# speedup vs baseline: 1.4027x; 1.0485x over previous
import functools

import jax
import jax.numpy as jnp
from jax import lax
from jax.experimental import pallas as pl
from jax.experimental.pallas import tpu as pltpu


def _vq_body(x_ref, w_ref, q_ref, loss_ref, iota_ref, w2r_ref, *, C):
    i = pl.program_id(0)
    xp = x_ref[0]
    xr = xp.T
    w = w_ref[...]

    @pl.when(i == 0)
    def _():
        iota_ref[...] = lax.broadcasted_iota(jnp.int32, iota_ref.shape, 1)
        w2 = jnp.sum(w * w, axis=1, keepdims=True)
        w2r_ref[...] = w2.T

    x2 = jnp.sum(xr * xr, axis=1, keepdims=True)
    s2 = lax.dot_general(xr, w + w, (((1,), (1,)), ((), ())),
                         preferred_element_type=jnp.float32)
    w2r = w2r_ref[...]
    d2 = jnp.maximum(x2 - s2 + w2r, 0.0)
    dist = jnp.sqrt(d2)

    minv = jnp.min(dist, axis=1, keepdims=True)
    iota = iota_ref[...]
    idx = jnp.min(jnp.where(dist == minv, iota, C), axis=1,
                  keepdims=True)

    onehot = (iota == idx).astype(jnp.float32)
    q = lax.dot_general(onehot, w, (((1,), (0,)), ((), ())),
                        preferred_element_type=jnp.float32)

    qst = xr + (q - xr)
    q_ref[0] = qst.T

    diff = q - xr
    part = jnp.sum(diff * diff)

    @pl.when(i == 0)
    def _():
        loss_ref[0, 0] = 0.0

    loss_ref[0, 0] += part


def kernel(inputs, embeddings_weight):
    N, D, H, W = inputs.shape
    C = embeddings_weight.shape[0]
    P = H * W
    x3 = inputs.reshape(N, D, P)

    q3, loss_sum = pl.pallas_call(
        functools.partial(_vq_body, C=C),
        grid=(N,),
        in_specs=[
            pl.BlockSpec((1, D, P), lambda i: (i, 0, 0)),
            pl.BlockSpec((C, D), lambda i: (0, 0)),
        ],
        out_specs=[
            pl.BlockSpec((1, D, P), lambda i: (i, 0, 0)),
            pl.BlockSpec((1, 1), lambda i: (0, 0), memory_space=pltpu.SMEM),
        ],
        out_shape=[
            jax.ShapeDtypeStruct((N, D, P), jnp.float32),
            jax.ShapeDtypeStruct((1, 1), jnp.float32),
        ],
        scratch_shapes=[
            pltpu.VMEM((P, C), jnp.int32),
            pltpu.VMEM((1, C), jnp.float32),
        ],
    )(x3, embeddings_weight)

    quantized_st = q3.reshape(N, D, H, W)
    c_loss = loss_sum[0, 0] * jnp.float32(1.25) / jnp.float32(N * D * H * W)
    return (c_loss, quantized_st)
